# Initial kernel scaffold; baseline (speedup 1.0000x reference)
import jax
import jax.numpy as jnp
from jax.experimental import pallas as pl


def kernel(x, edge_index, W1, b1, W2, b2, W3, b3):
    raise NotImplementedError("write your pallas kernel here")


# speedup vs baseline: 10.6234x; 10.6234x over previous
import functools

import jax
import jax.numpy as jnp
from jax.experimental import pallas as pl
from jax.experimental.pallas import tpu as pltpu
from jax.experimental.pallas import tpu_sc as plsc

_N = 10000
_E = 320000
_D = 128
_NC = 2
_NS = 16
_NW = _NC * _NS
_EPW = _E // _NW
_K = 80
_RPT = 624
_REM_BASE = _NS * _RPT
_REM = _N - _REM_BASE

_BLK = 1000
_G = _N // _BLK

_sc_mesh = functools.partial(
    plsc.VectorSubcoreMesh, core_axis_name="c", subcore_axis_name="s"
)


def _sc_degree(dst, ones_rows, zeros16):
    @functools.partial(
        pl.kernel,
        out_type=jax.ShapeDtypeStruct((_NC, _N, _D), jnp.float32),
        mesh=_sc_mesh(),
        scratch_types=[
            pltpu.VMEM((_K,), jnp.int32),
            pltpu.VMEM((_K, _D), jnp.float32),
            pltpu.VMEM_SHARED((_N, _D), jnp.float32),
        ],
    )
    def k(dst_hbm, ones_hbm, zero_hbm, out_hbm, didx, ones_v, acc):
        c = jax.lax.axis_index("c")
        s = jax.lax.axis_index("s")
        wid = c * _NS + s
        r0 = s * _RPT
        pltpu.sync_copy(ones_hbm, ones_v)
        pltpu.sync_copy(zero_hbm.at[pl.ds(r0, _RPT)], acc.at[pl.ds(r0, _RPT)])

        @pl.when(s == 0)
        def _():
            pltpu.sync_copy(zero_hbm.at[pl.ds(_REM_BASE, _REM)],
                            acc.at[pl.ds(_REM_BASE, _REM)])

        plsc.subcore_barrier()
        base = wid * _EPW

        @pl.loop(0, _EPW, step=_K)
        def _(off):
            pltpu.sync_copy(dst_hbm.at[pl.ds(base + off, _K)], didx)
            pltpu.sync_copy(ones_v, acc.at[didx], add=True)

        plsc.subcore_barrier()
        pltpu.sync_copy(acc.at[pl.ds(r0, _RPT)], out_hbm.at[c, pl.ds(r0, _RPT)])

        @pl.when(s == 0)
        def _():
            pltpu.sync_copy(acc.at[pl.ds(_REM_BASE, _REM)],
                            out_hbm.at[c, pl.ds(_REM_BASE, _REM)])

    return k(dst, ones_rows, zeros16)


def _sc_propagate(hs, src, dst, zeros):
    @functools.partial(
        pl.kernel,
        out_type=jax.ShapeDtypeStruct((_NC, _N, _D), jnp.float32),
        mesh=_sc_mesh(),
        scratch_types=[
            pltpu.VMEM((_K,), jnp.int32),
            pltpu.VMEM((_K,), jnp.int32),
            pltpu.VMEM((_K, _D), jnp.float32),
            pltpu.VMEM_SHARED((_N, _D), jnp.float32),
            pltpu.SemaphoreType.DMA,
        ],
    )
    def k(hs_hbm, src_hbm, dst_hbm, zero_hbm, out_hbm, sidx, didx, rows, acc, sem):
        c = jax.lax.axis_index("c")
        s = jax.lax.axis_index("s")
        wid = c * _NS + s
        r0 = s * _RPT
        pltpu.sync_copy(zero_hbm.at[pl.ds(r0, _RPT)], acc.at[pl.ds(r0, _RPT)])

        @pl.when(s == 0)
        def _():
            pltpu.sync_copy(zero_hbm.at[pl.ds(_REM_BASE, _REM)],
                            acc.at[pl.ds(_REM_BASE, _REM)])

        plsc.subcore_barrier()
        base = wid * _EPW

        @pl.loop(0, _EPW, step=_K)
        def _(off):
            pltpu.sync_copy(src_hbm.at[pl.ds(base + off, _K)], sidx)
            pltpu.sync_copy(dst_hbm.at[pl.ds(base + off, _K)], didx)
            pltpu.async_copy(hs_hbm.at[sidx], rows, sem).wait()
            pltpu.sync_copy(rows, acc.at[didx], add=True)

        plsc.subcore_barrier()
        pltpu.sync_copy(acc.at[pl.ds(r0, _RPT)], out_hbm.at[c, pl.ds(r0, _RPT)])

        @pl.when(s == 0)
        def _():
            pltpu.sync_copy(acc.at[pl.ds(_REM_BASE, _REM)],
                            out_hbm.at[c, pl.ds(_REM_BASE, _REM)])

    return k(hs, src, dst, zeros)


def _dis_block(degp):
    deg = 1.0 + degp[0, :, 0:1] + degp[1, :, 0:1]
    return jax.lax.rsqrt(deg), deg


_row_spec = pl.BlockSpec((_BLK, _D), lambda i: (i, 0))
_p_spec = pl.BlockSpec((_NC, _BLK, _D), lambda i: (0, i, 0))
_deg_spec = pl.BlockSpec((_NC, _BLK, _D), lambda i: (0, i, 0))
_w_spec = pl.BlockSpec((_D, _D), lambda i: (0, 0))
_b_spec = pl.BlockSpec((1, _D), lambda i: (0, 0))


def _tc_mm(x, W1):
    def body(x_ref, w_ref, h_ref):
        h_ref[...] = jnp.dot(x_ref[...], w_ref[...],
                             preferred_element_type=jnp.float32)

    return pl.pallas_call(
        body,
        grid=(_G,),
        in_specs=[_row_spec, _w_spec],
        out_specs=_row_spec,
        out_shape=jax.ShapeDtypeStruct((_N, _D), jnp.float32),
    )(x, W1)


def _tc_scale(h, degp):
    def body(h_ref, degp_ref, hs_ref):
        dis, _ = _dis_block(degp_ref[...])
        hs_ref[...] = h_ref[...] * dis

    return pl.pallas_call(
        body,
        grid=(_G,),
        in_specs=[_row_spec, _deg_spec],
        out_specs=_row_spec,
        out_shape=jax.ShapeDtypeStruct((_N, _D), jnp.float32),
    )(h, degp)


def _tc_mid(p, h, degp, b, res, Wn):
    have_res = res is not None

    def body(*refs):
        if have_res:
            p_ref, h_ref, degp_ref, b_ref, res_ref, w_ref, a_ref, hn_ref, hs_ref = refs
        else:
            p_ref, h_ref, degp_ref, b_ref, w_ref, a_ref, hn_ref, hs_ref = refs
        dis, deg = _dis_block(degp_ref[...])
        agg = p_ref[0] + p_ref[1]
        a = dis * agg + h_ref[...] / deg + b_ref[...]
        if have_res:
            a = a + res_ref[...]
        a = jnp.maximum(a, 0.0)
        hn = jnp.dot(a, w_ref[...], preferred_element_type=jnp.float32)
        a_ref[...] = a
        hn_ref[...] = hn
        hs_ref[...] = hn * dis

    in_specs = [_p_spec, _row_spec, _deg_spec, _b_spec]
    args = [p, h, degp, b.reshape(1, _D)]
    if have_res:
        in_specs.append(_row_spec)
        args.append(res)
    in_specs.append(_w_spec)
    args.append(Wn)
    return pl.pallas_call(
        body,
        grid=(_G,),
        in_specs=in_specs,
        out_specs=[_row_spec, _row_spec, _row_spec],
        out_shape=[
            jax.ShapeDtypeStruct((_N, _D), jnp.float32),
            jax.ShapeDtypeStruct((_N, _D), jnp.float32),
            jax.ShapeDtypeStruct((_N, _D), jnp.float32),
        ],
    )(*args)


def _tc_final(p, h, degp, b):
    def body(p_ref, h_ref, degp_ref, b_ref, o_ref):
        dis, deg = _dis_block(degp_ref[...])
        a = dis * (p_ref[0] + p_ref[1]) + h_ref[...] / deg + b_ref[...]
        o_ref[...] = jax.nn.sigmoid(a)

    return pl.pallas_call(
        body,
        grid=(_G,),
        in_specs=[_p_spec, _row_spec, _deg_spec, _b_spec],
        out_specs=_row_spec,
        out_shape=jax.ShapeDtypeStruct((_N, _D), jnp.float32),
    )(p, h, degp, b.reshape(1, _D))


def kernel(x, edge_index, W1, b1, W2, b2, W3, b3):
    src = edge_index[0]
    dst = edge_index[1]
    zeros = jnp.zeros((_N, _D), jnp.float32)
    ones_rows = jnp.ones((_K, _D), jnp.float32)

    degp = _sc_degree(dst, ones_rows, zeros)
    h1 = _tc_mm(x, W1)
    hs1 = _tc_scale(h1, degp)
    p1 = _sc_propagate(hs1, src, dst, zeros)

    act1, h2, hs2 = _tc_mid(p1, h1, degp, b1, None, W2)
    p2 = _sc_propagate(hs2, src, dst, zeros)

    _, h3, hs3 = _tc_mid(p2, h2, degp, b2, act1, W3)
    p3 = _sc_propagate(hs3, src, dst, zeros)

    return _tc_final(p3, h3, degp, b3)
